# baseline (speedup 1.0000x reference)
import jax
import jax.numpy as jnp
from jax import lax
from jax.experimental import pallas as pl
from jax.experimental.pallas import tpu as pltpu

_BN = 1024


def _proj_body(hl_ref, W2_ref, b2_ref, proj_ref):
    proj = lax.dot_general(hl_ref[...], W2_ref[...],
                           (((1,), (1,)), ((), ())),
                           precision=lax.Precision.HIGHEST,
                           preferred_element_type=jnp.float32)
    proj_ref[...] = proj + b2_ref[...]


def _stream_body(scal_ref, proj_ref, vW_ref, g_ref, q_ref,
                 logits_ref, idx_ref, m_ref, mi_ref):
    i = pl.program_id(0)
    nb = pl.num_programs(0)

    temp = scal_ref[0, 0]
    tanh_c = scal_ref[0, 1]
    vb_c = scal_ref[0, 2]

    hidden = jnp.tanh(q_ref[...] + proj_ref[...])
    col = jnp.sum(hidden * vW_ref[...], axis=1, keepdims=True)
    logits_blk = tanh_c * jnp.tanh((col + vb_c) / temp)
    logits_ref[...] = logits_blk

    score = logits_blk + g_ref[...]
    local_max = jnp.max(score)
    ids = lax.broadcasted_iota(jnp.int32, score.shape, 0)
    local_arg = jnp.min(jnp.where(score == local_max, ids, score.shape[0]))

    @pl.when(i == 0)
    def _init():
        m_ref[0] = -jnp.inf
        mi_ref[0] = 0

    cur_m = m_ref[0]
    upd = local_max > cur_m
    m_ref[0] = jnp.where(upd, local_max, cur_m)
    mi_ref[0] = jnp.where(upd, i * score.shape[0] + local_arg, mi_ref[0])

    @pl.when(i == nb - 1)
    def _fin():
        idx_ref[0, 0] = mi_ref[0]


def kernel(h, query, W2, b2, vW, vb, temperature, tanh_constant):
    N, L = query.shape
    bn = min(_BN, N)
    nb = N // bn
    hl = h[-1].reshape(1, L)
    g = jax.random.gumbel(jax.random.key(42), (1, N), jnp.float32).reshape(N, 1)
    scal = jnp.stack([jnp.asarray(temperature, jnp.float32),
                      jnp.asarray(tanh_constant, jnp.float32),
                      vb.astype(jnp.float32)[0],
                      jnp.float32(0)]).reshape(1, 4)

    proj = pl.pallas_call(
        _proj_body,
        out_shape=jax.ShapeDtypeStruct((1, L), jnp.float32),
    )(hl, W2, b2.reshape(1, L))

    logits_col, idx = pl.pallas_call(
        _stream_body,
        grid=(nb,),
        in_specs=[
            pl.BlockSpec(memory_space=pltpu.SMEM),
            pl.BlockSpec((1, L), lambda i: (0, 0)),
            pl.BlockSpec((1, L), lambda i: (0, 0)),
            pl.BlockSpec((bn, 1), lambda i: (i, 0)),
            pl.BlockSpec((bn, L), lambda i: (i, 0)),
        ],
        out_specs=[
            pl.BlockSpec((bn, 1), lambda i: (i, 0)),
            pl.BlockSpec((1, 1), lambda i: (0, 0),
                         memory_space=pltpu.SMEM),
        ],
        out_shape=[
            jax.ShapeDtypeStruct((N, 1), jnp.float32),
            jax.ShapeDtypeStruct((1, 1), jnp.int32),
        ],
        scratch_shapes=[
            pltpu.SMEM((1,), jnp.float32),
            pltpu.SMEM((1,), jnp.int32),
        ],
    )(scal, proj, vW, g, query)

    return (idx[0, 0], logits_col.reshape(1, N))


# speedup vs baseline: 1.0709x; 1.0709x over previous
import jax
import jax.numpy as jnp
from jax import lax
from jax.experimental import pallas as pl
from jax.experimental.pallas import tpu as pltpu

_BN = 2048


def _proj_body(hl_ref, W2_ref, b2_ref, proj_ref):
    proj = lax.dot_general(hl_ref[...], W2_ref[...],
                           (((1,), (1,)), ((), ())),
                           precision=lax.Precision.HIGHEST,
                           preferred_element_type=jnp.float32)
    proj_ref[...] = proj + b2_ref[...]


def _stream_body(scal_ref, proj_ref, vW_ref, g_ref, q_ref,
                 logits_ref, idx_ref, m_ref, mi_ref):
    i = pl.program_id(0)
    nb = pl.num_programs(0)

    temp = scal_ref[0, 0]
    tanh_c = scal_ref[0, 1]
    vb_c = scal_ref[0, 2]

    hidden = jnp.tanh(q_ref[...] + proj_ref[...])
    col = jnp.sum(hidden * vW_ref[...], axis=1, keepdims=True)
    logits_blk = tanh_c * jnp.tanh((col + vb_c) / temp)
    logits_ref[...] = logits_blk

    score = logits_blk + g_ref[...]
    local_max = jnp.max(score)
    ids = lax.broadcasted_iota(jnp.int32, score.shape, 0)
    local_arg = jnp.min(jnp.where(score == local_max, ids, score.shape[0]))

    @pl.when(i == 0)
    def _init():
        m_ref[0] = -jnp.inf
        mi_ref[0] = 0

    cur_m = m_ref[0]
    upd = local_max > cur_m
    m_ref[0] = jnp.where(upd, local_max, cur_m)
    mi_ref[0] = jnp.where(upd, i * score.shape[0] + local_arg, mi_ref[0])

    @pl.when(i == nb - 1)
    def _fin():
        idx_ref[0, 0] = mi_ref[0]


def kernel(h, query, W2, b2, vW, vb, temperature, tanh_constant):
    N, L = query.shape
    bn = min(_BN, N)
    nb = N // bn
    hl = h[-1].reshape(1, L)
    g = jax.random.gumbel(jax.random.key(42), (1, N), jnp.float32).reshape(N, 1)
    scal = jnp.stack([jnp.asarray(temperature, jnp.float32),
                      jnp.asarray(tanh_constant, jnp.float32),
                      vb.astype(jnp.float32)[0],
                      jnp.float32(0)]).reshape(1, 4)

    proj = pl.pallas_call(
        _proj_body,
        out_shape=jax.ShapeDtypeStruct((1, L), jnp.float32),
    )(hl, W2, b2.reshape(1, L))

    logits_col, idx = pl.pallas_call(
        _stream_body,
        grid=(nb,),
        in_specs=[
            pl.BlockSpec(memory_space=pltpu.SMEM),
            pl.BlockSpec((1, L), lambda i: (0, 0)),
            pl.BlockSpec((1, L), lambda i: (0, 0)),
            pl.BlockSpec((bn, 1), lambda i: (i, 0)),
            pl.BlockSpec((bn, L), lambda i: (i, 0)),
        ],
        out_specs=[
            pl.BlockSpec((bn, 1), lambda i: (i, 0)),
            pl.BlockSpec((1, 1), lambda i: (0, 0),
                         memory_space=pltpu.SMEM),
        ],
        out_shape=[
            jax.ShapeDtypeStruct((N, 1), jnp.float32),
            jax.ShapeDtypeStruct((1, 1), jnp.int32),
        ],
        scratch_shapes=[
            pltpu.SMEM((1,), jnp.float32),
            pltpu.SMEM((1,), jnp.int32),
        ],
    )(scal, proj, vW, g, query)

    return (idx[0, 0], logits_col.reshape(1, N))
